# baseline (speedup 1.0000x reference)
import functools

import jax
import jax.numpy as jnp
from jax import lax
from jax.experimental import pallas as pl
from jax.experimental.pallas import tpu as pltpu
from jax.experimental.pallas import tpu_sc as plsc

N = 10000
E = 320000
DF = 128
M = 10240
NCORE = 2
NSUB = 16
NT = NCORE * NSUB
EP = E // NT
EPP = 10240
CS = 80
NCS = EPP // CS
ROWS_PER_TILE = M // NSUB


def _sc_row_aggregate_body(x_hbm, src_hbm, dst_hbm, zeros_hbm, out_hbm,
                           src_v, dst_v, rows_v, acc_sh, sem):
    c = lax.axis_index("c")
    s = lax.axis_index("s")
    w = c * NSUB + s

    @pl.when(s == 0)
    def _():
        pltpu.sync_copy(zeros_hbm, acc_sh)

    pltpu.sync_copy(src_hbm.at[w], src_v)
    pltpu.sync_copy(dst_hbm.at[w], dst_v)
    plsc.subcore_barrier()

    def body(j, carry):
        pltpu.async_copy(x_hbm.at[src_v.at[j]], rows_v, sem).wait()
        pltpu.sync_copy(rows_v, acc_sh.at[dst_v.at[j]], add=True)
        return carry

    lax.fori_loop(0, NCS, body, 0, unroll=False)
    plsc.subcore_barrier()
    pltpu.sync_copy(acc_sh.at[pl.ds(s * ROWS_PER_TILE, ROWS_PER_TILE)],
                    out_hbm.at[c, pl.ds(s * ROWS_PER_TILE, ROWS_PER_TILE)])


@functools.lru_cache(maxsize=None)
def _sc_row_aggregate():
    mesh = plsc.VectorSubcoreMesh(core_axis_name="c", subcore_axis_name="s",
                                  num_cores=NCORE, num_subcores=NSUB)
    return pl.kernel(
        _sc_row_aggregate_body,
        out_type=jax.ShapeDtypeStruct((NCORE, M, DF), jnp.float32),
        mesh=mesh,
        scratch_types=[
            pltpu.VMEM((NCS, CS), jnp.int32),
            pltpu.VMEM((NCS, CS), jnp.int32),
            pltpu.VMEM((CS, DF), jnp.float32),
            pltpu.VMEM_SHARED((M, DF), jnp.float32),
            pltpu.SemaphoreType.DMA,
        ],
        compiler_params=pltpu.CompilerParams(needs_layout_passes=False),
    )


def _sc_segsum_body(xs_hbm, src_hbm, dst_hbm, zeros_hbm, out_hbm,
                    xs_v, src_v, dst_v, vals_v, acc_sh, sem):
    c = lax.axis_index("c")
    s = lax.axis_index("s")
    w = c * NSUB + s

    @pl.when(s == 0)
    def _():
        pltpu.sync_copy(zeros_hbm, acc_sh)

    pltpu.sync_copy(xs_hbm, xs_v)
    pltpu.sync_copy(src_hbm.at[w], src_v)
    pltpu.sync_copy(dst_hbm.at[w], dst_v)
    plsc.subcore_barrier()

    def gbody(j, carry):
        for k in range(CS // 16):
            idx = src_v[j, pl.ds(k * 16, 16)]
            vals_v[pl.ds(j * CS + k * 16, 16)] = plsc.load_gather(xs_v, [idx])
        return carry

    lax.fori_loop(0, NCS, gbody, 0, unroll=False)

    FIRE = 8

    def sbody(jo, carry):
        base = jo * FIRE
        cps = [
            pltpu.async_copy(vals_v.at[pl.ds((base + k) * CS, CS)],
                             acc_sh.at[dst_v.at[base + k]], sem, add=True)
            for k in range(FIRE)
        ]
        for cp in cps:
            cp.wait()
        return carry

    lax.fori_loop(0, NCS // FIRE, sbody, 0, unroll=False)
    plsc.subcore_barrier()
    pltpu.sync_copy(acc_sh.at[pl.ds(s * ROWS_PER_TILE, ROWS_PER_TILE)],
                    out_hbm.at[c, pl.ds(s * ROWS_PER_TILE, ROWS_PER_TILE)])


@functools.lru_cache(maxsize=None)
def _sc_segsum():
    mesh = plsc.VectorSubcoreMesh(core_axis_name="c", subcore_axis_name="s",
                                  num_cores=NCORE, num_subcores=NSUB)
    return pl.kernel(
        _sc_segsum_body,
        out_type=jax.ShapeDtypeStruct((NCORE, M), jnp.float32),
        mesh=mesh,
        scratch_types=[
            pltpu.VMEM((M,), jnp.float32),
            pltpu.VMEM((NCS, CS), jnp.int32),
            pltpu.VMEM((NCS, CS), jnp.int32),
            pltpu.VMEM((EPP,), jnp.float32),
            pltpu.VMEM_SHARED((M,), jnp.float32),
            pltpu.SemaphoreType.DMA,
        ],
        compiler_params=pltpu.CompilerParams(needs_layout_passes=False),
    )


def _p0_body(w1p_ref, x_ref, a0_ref, a1_ref, o_ref):
    xsum = x_ref[...] + a0_ref[...] + a1_ref[...]
    o_ref[...] = lax.dot_general(
        w1p_ref[...], xsum, (((0,), (1,)), ((), ())),
        preferred_element_type=jnp.float32, precision=lax.Precision.HIGHEST)


NB0 = 2048
_p0 = pl.pallas_call(
    _p0_body,
    grid=(M // NB0,),
    in_specs=[
        pl.BlockSpec((DF, DF), lambda i: (0, 0)),
        pl.BlockSpec((NB0, DF), lambda i: (i, 0)),
        pl.BlockSpec((NB0, DF), lambda i: (i, 0)),
        pl.BlockSpec((NB0, DF), lambda i: (i, 0)),
    ],
    out_specs=pl.BlockSpec((DF, NB0), lambda i: (0, i)),
    out_shape=jax.ShapeDtypeStruct((DF, M), jnp.float32),
)


def _finish_step(tau, hsumT, xs, sarr, sol, wl, w2,
                 pred_ref, label_ref, dir_ref, xs_ref, s_ref):
    col = lax.broadcasted_iota(jnp.int32, (1, M), 1)
    u = xs + sarr
    zt = jnp.tanh(hsumT + wl * u)
    pred = jnp.sum(zt * w2, axis=0, keepdims=True)
    pred = jnp.where(col < N, pred, 0.0)
    t1 = jnp.sum(jnp.abs(pred))
    res = sol - xs
    t2 = jnp.sum(jnp.abs(res))
    pred_ref[...] = pred
    label_ref[...] = res / (t2 + 1e-8)
    dir_ref[...] = pred / (t1 + 1e-8) + (3.0 * tau) / (xs + tau)
    xs_ref[...] = xs
    s_ref[...] = sarr


def _p1a_body(tau, hsumT_ref, xs_ref, s0_ref, s1_ref, sol_ref, wl_ref,
              w2_ref, pred_o, label_o, dir_o, xs_o, s_o):
    _finish_step(tau, hsumT_ref[...], xs_ref[...], s0_ref[...] + s1_ref[...],
                 sol_ref[...], wl_ref[...], w2_ref[...],
                 pred_o, label_o, dir_o, xs_o, s_o)


def _p1b_body(tau, hsumT_ref, xsp_ref, sprev_ref, q0_ref, q1_ref, pp_ref,
              sol_ref, wl_ref, w2_ref, pred_o, label_o, dir_o, xs_o, s_o):
    col = lax.broadcasted_iota(jnp.int32, (1, M), 1)
    valid = col < N
    xsp = xsp_ref[...]
    pp = pp_ref[...]
    ratios = jnp.where(valid & (pp < 0.0),
                       xsp / jnp.maximum(-pp, 1e-12), jnp.inf)
    alpha = jnp.minimum(jnp.min(ratios), STEP_ALPHA_MAX) * 0.995
    xs = jnp.where(valid, xsp + alpha * pp, 0.0)
    sarr = sprev_ref[...] + alpha * (q0_ref[...] + q1_ref[...])
    _finish_step(tau, hsumT_ref[...], xs, sarr, sol_ref[...], wl_ref[...],
                 w2_ref[...], pred_o, label_o, dir_o, xs_o, s_o)


STEP_ALPHA_MAX = 5.0
_P1_OUT = [jax.ShapeDtypeStruct((1, M), jnp.float32)] * 5


def _make_p1a(tau):
    return pl.pallas_call(functools.partial(_p1a_body, tau),
                          out_shape=_P1_OUT)


def _make_p1b(tau):
    return pl.pallas_call(functools.partial(_p1b_body, tau),
                          out_shape=_P1_OUT)


_TAUS = []
_t = 0.01
for _ in range(4):
    _TAUS.append(_t)
    _t = max(_t * 0.5, 1e-5)
_p1a_call = _make_p1a(_TAUS[0])
_p1b_calls = [None] + [_make_p1b(t) for t in _TAUS[1:]]


def _p2_body(p_ref, d_ref, o_ref):
    o_ref[...] = lax.dot_general(
        p_ref[...], d_ref[...], (((1,), (0,)), ((), ())),
        preferred_element_type=jnp.float32, precision=lax.Precision.HIGHEST)


RB = 256
_p2 = pl.pallas_call(
    _p2_body,
    grid=(M // RB,),
    in_specs=[
        pl.BlockSpec((RB, N), lambda i: (i, 0)),
        pl.BlockSpec((N,), lambda i: (0,)),
    ],
    out_specs=pl.BlockSpec((RB,), lambda i: (i,)),
    out_shape=jax.ShapeDtypeStruct((M,), jnp.float32),
)


def kernel(x, x_start, x_solution, proj_matrix, W1, W2, edge_index, vals_batch):
    del vals_batch
    f32 = jnp.float32
    x = x.astype(f32)
    xp = jnp.pad(x, ((0, M - N), (0, 0)))
    xsp = jnp.pad(x_start.astype(f32), (0, M - N))
    solp = jnp.pad(x_solution.astype(f32), (0, M - N)).reshape(1, M)
    W1p = W1[:DF]
    wl = W1[DF].reshape(DF, 1)
    w2 = W2.reshape(DF, 1)
    srcp = jnp.concatenate(
        [edge_index[0].reshape(NT, EP),
         jnp.zeros((NT, EPP - EP), jnp.int32)], axis=1)
    dstp = jnp.concatenate(
        [edge_index[1].reshape(NT, EP),
         jnp.broadcast_to(N + jnp.arange(EPP - EP, dtype=jnp.int32),
                          (NT, EPP - EP))], axis=1)
    src2 = srcp.reshape(NT, NCS, CS)
    dst2 = dstp.reshape(NT, NCS, CS)
    zrows = jnp.zeros((M, DF), f32)
    zvec = jnp.zeros((M,), f32)

    ax = _sc_row_aggregate()(x, src2, dst2, zrows)
    hsumT = _p0(W1p, xp, ax[0], ax[1])

    s0p = _sc_segsum()(xsp, src2, dst2, zvec)
    pred, label, direc, xs2d, scomb = _p1a_call(
        hsumT, xsp.reshape(1, M), s0p[0].reshape(1, M), s0p[1].reshape(1, M),
        solp, wl, w2)
    preds = [pred[0, :N]]
    labels = [label[0, :N]]
    for t in range(1, 4):
        ppad = _p2(proj_matrix, direc[0, :N])
        qp = _sc_segsum()(ppad, src2, dst2, zvec)
        pred, label, direc, xs2d, scomb = _p1b_calls[t](
            hsumT, xs2d, scomb, qp[0].reshape(1, M), qp[1].reshape(1, M),
            ppad.reshape(1, M), solp, wl, w2)
        preds.append(pred[0, :N])
        labels.append(label[0, :N])
    return jnp.stack(preds, 1), jnp.stack(labels, 1)


# speedup vs baseline: 1.3916x; 1.3916x over previous
import functools

import jax
import jax.numpy as jnp
from jax import lax
from jax.experimental import pallas as pl
from jax.experimental.pallas import tpu as pltpu
from jax.experimental.pallas import tpu_sc as plsc

N = 10000
E = 320000
DF = 128
M = 10240
NCORE = 2
NSUB = 16
NT = NCORE * NSUB
EP = E // NT
CS = 80
NCS = EP // CS
ROWS_PER_TILE = M // NSUB


def _sc_row_aggregate_body(x_hbm, src_hbm, dst_hbm, zeros_hbm, out_hbm,
                           src_v, dst_v, rows_v, acc_sh, sem):
    c = lax.axis_index("c")
    s = lax.axis_index("s")
    w = c * NSUB + s

    @pl.when(s == 0)
    def _():
        pltpu.sync_copy(zeros_hbm, acc_sh)

    pltpu.sync_copy(src_hbm.at[w], src_v)
    pltpu.sync_copy(dst_hbm.at[w], dst_v)
    plsc.subcore_barrier()

    def body(j, carry):
        pltpu.async_copy(x_hbm.at[src_v.at[j]], rows_v, sem).wait()
        pltpu.sync_copy(rows_v, acc_sh.at[dst_v.at[j]], add=True)
        return carry

    lax.fori_loop(0, NCS, body, 0, unroll=False)
    plsc.subcore_barrier()
    pltpu.sync_copy(acc_sh.at[pl.ds(s * ROWS_PER_TILE, ROWS_PER_TILE)],
                    out_hbm.at[c, pl.ds(s * ROWS_PER_TILE, ROWS_PER_TILE)])


@functools.lru_cache(maxsize=None)
def _sc_row_aggregate():
    mesh = plsc.VectorSubcoreMesh(core_axis_name="c", subcore_axis_name="s",
                                  num_cores=NCORE, num_subcores=NSUB)
    return pl.kernel(
        _sc_row_aggregate_body,
        out_type=jax.ShapeDtypeStruct((NCORE, M, DF), jnp.float32),
        mesh=mesh,
        scratch_types=[
            pltpu.VMEM((NCS, CS), jnp.int32),
            pltpu.VMEM((NCS, CS), jnp.int32),
            pltpu.VMEM((CS, DF), jnp.float32),
            pltpu.VMEM_SHARED((M, DF), jnp.float32),
            pltpu.SemaphoreType.DMA,
        ],
        compiler_params=pltpu.CompilerParams(needs_layout_passes=False),
    )


def _sc_segsum_body(xs_hbm, src_hbm, dst_hbm, zeros_hbm, out_hbm,
                    xs_v, src_v, dst_v, vals_v, acc_sh, sem):
    c = lax.axis_index("c")
    s = lax.axis_index("s")
    w = c * NSUB + s

    @pl.when(s == 0)
    def _():
        pltpu.sync_copy(zeros_hbm, acc_sh)

    pltpu.sync_copy(xs_hbm, xs_v)
    pltpu.sync_copy(src_hbm.at[w], src_v)
    pltpu.sync_copy(dst_hbm.at[w], dst_v)
    plsc.subcore_barrier()

    def gbody(j, carry):
        for k in range(CS // 16):
            idx = src_v[j, pl.ds(k * 16, 16)]
            vals_v[pl.ds(j * CS + k * 16, 16)] = plsc.load_gather(xs_v, [idx])
        return carry

    lax.fori_loop(0, NCS, gbody, 0, unroll=False)

    FIRE = 5

    def sbody(jo, carry):
        base = jo * FIRE
        cps = [
            pltpu.async_copy(vals_v.at[pl.ds((base + k) * CS, CS)],
                             acc_sh.at[dst_v.at[base + k]], sem, add=True)
            for k in range(FIRE)
        ]
        for cp in cps:
            cp.wait()
        return carry

    lax.fori_loop(0, NCS // FIRE, sbody, 0, unroll=False)
    plsc.subcore_barrier()
    pltpu.sync_copy(acc_sh.at[pl.ds(s * ROWS_PER_TILE, ROWS_PER_TILE)],
                    out_hbm.at[c, pl.ds(s * ROWS_PER_TILE, ROWS_PER_TILE)])


@functools.lru_cache(maxsize=None)
def _sc_segsum():
    mesh = plsc.VectorSubcoreMesh(core_axis_name="c", subcore_axis_name="s",
                                  num_cores=NCORE, num_subcores=NSUB)
    return pl.kernel(
        _sc_segsum_body,
        out_type=jax.ShapeDtypeStruct((NCORE, M), jnp.float32),
        mesh=mesh,
        scratch_types=[
            pltpu.VMEM((M,), jnp.float32),
            pltpu.VMEM((NCS, CS), jnp.int32),
            pltpu.VMEM((NCS, CS), jnp.int32),
            pltpu.VMEM((EP,), jnp.float32),
            pltpu.VMEM_SHARED((M,), jnp.float32),
            pltpu.SemaphoreType.DMA,
        ],
        compiler_params=pltpu.CompilerParams(needs_layout_passes=False),
    )


def _p0_body(w1p_ref, x_ref, a0_ref, a1_ref, o_ref):
    xsum = x_ref[...] + a0_ref[...] + a1_ref[...]
    o_ref[...] = lax.dot_general(
        w1p_ref[...], xsum, (((0,), (1,)), ((), ())),
        preferred_element_type=jnp.float32, precision=lax.Precision.HIGHEST)


NB0 = 2048
_p0 = pl.pallas_call(
    _p0_body,
    grid=(M // NB0,),
    in_specs=[
        pl.BlockSpec((DF, DF), lambda i: (0, 0)),
        pl.BlockSpec((NB0, DF), lambda i: (i, 0)),
        pl.BlockSpec((NB0, DF), lambda i: (i, 0)),
        pl.BlockSpec((NB0, DF), lambda i: (i, 0)),
    ],
    out_specs=pl.BlockSpec((DF, NB0), lambda i: (0, i)),
    out_shape=jax.ShapeDtypeStruct((DF, M), jnp.float32),
)


def _finish_step(tau, hsumT, xs, sarr, sol, wl, w2,
                 pred_ref, label_ref, dir_ref, xs_ref, s_ref):
    col = lax.broadcasted_iota(jnp.int32, (1, M), 1)
    u = xs + sarr
    zt = jnp.tanh(hsumT + wl * u)
    pred = jnp.sum(zt * w2, axis=0, keepdims=True)
    pred = jnp.where(col < N, pred, 0.0)
    t1 = jnp.sum(jnp.abs(pred))
    res = sol - xs
    t2 = jnp.sum(jnp.abs(res))
    pred_ref[...] = pred
    label_ref[...] = res / (t2 + 1e-8)
    dir_ref[...] = pred / (t1 + 1e-8) + (3.0 * tau) / (xs + tau)
    xs_ref[...] = xs
    s_ref[...] = sarr


def _p1a_body(tau, hsumT_ref, xs_ref, s0_ref, s1_ref, sol_ref, wl_ref,
              w2_ref, pred_o, label_o, dir_o, xs_o, s_o):
    _finish_step(tau, hsumT_ref[...], xs_ref[...], s0_ref[...] + s1_ref[...],
                 sol_ref[...], wl_ref[...], w2_ref[...],
                 pred_o, label_o, dir_o, xs_o, s_o)


def _p1b_body(tau, hsumT_ref, xsp_ref, sprev_ref, q0_ref, q1_ref, pp_ref,
              sol_ref, wl_ref, w2_ref, pred_o, label_o, dir_o, xs_o, s_o):
    col = lax.broadcasted_iota(jnp.int32, (1, M), 1)
    valid = col < N
    xsp = xsp_ref[...]
    pp = pp_ref[...]
    ratios = jnp.where(valid & (pp < 0.0),
                       xsp / jnp.maximum(-pp, 1e-12), jnp.inf)
    alpha = jnp.minimum(jnp.min(ratios), STEP_ALPHA_MAX) * 0.995
    xs = jnp.where(valid, xsp + alpha * pp, 0.0)
    sarr = sprev_ref[...] + alpha * (q0_ref[...] + q1_ref[...])
    _finish_step(tau, hsumT_ref[...], xs, sarr, sol_ref[...], wl_ref[...],
                 w2_ref[...], pred_o, label_o, dir_o, xs_o, s_o)


STEP_ALPHA_MAX = 5.0
_P1_OUT = [jax.ShapeDtypeStruct((1, M), jnp.float32)] * 5


def _make_p1a(tau):
    return pl.pallas_call(functools.partial(_p1a_body, tau),
                          out_shape=_P1_OUT)


def _make_p1b(tau):
    return pl.pallas_call(functools.partial(_p1b_body, tau),
                          out_shape=_P1_OUT)


_TAUS = []
_t = 0.01
for _ in range(4):
    _TAUS.append(_t)
    _t = max(_t * 0.5, 1e-5)
_p1a_call = _make_p1a(_TAUS[0])
_p1b_calls = [None] + [_make_p1b(t) for t in _TAUS[1:]]


def _p2_body(p_ref, d_ref, o_ref):
    o_ref[...] = lax.dot_general(
        p_ref[...], d_ref[...], (((1,), (0,)), ((), ())),
        preferred_element_type=jnp.float32, precision=lax.Precision.HIGHEST)


RB = 256
_p2 = pl.pallas_call(
    _p2_body,
    grid=(M // RB,),
    in_specs=[
        pl.BlockSpec((RB, N), lambda i: (i, 0)),
        pl.BlockSpec((N,), lambda i: (0,)),
    ],
    out_specs=pl.BlockSpec((RB,), lambda i: (i,)),
    out_shape=jax.ShapeDtypeStruct((M,), jnp.float32),
)


def kernel(x, x_start, x_solution, proj_matrix, W1, W2, edge_index, vals_batch):
    del vals_batch
    f32 = jnp.float32
    x = x.astype(f32)
    xp = jnp.pad(x, ((0, M - N), (0, 0)))
    xsp = jnp.pad(x_start.astype(f32), (0, M - N))
    solp = jnp.pad(x_solution.astype(f32), (0, M - N)).reshape(1, M)
    W1p = W1[:DF]
    wl = W1[DF].reshape(DF, 1)
    w2 = W2.reshape(DF, 1)
    src2 = edge_index[0].reshape(NT, NCS, CS)
    dst2 = edge_index[1].reshape(NT, NCS, CS)
    zrows = jnp.zeros((M, DF), f32)
    zvec = jnp.zeros((M,), f32)

    ax = _sc_row_aggregate()(x, src2, dst2, zrows)
    hsumT = _p0(W1p, xp, ax[0], ax[1])

    s0p = _sc_segsum()(xsp, src2, dst2, zvec)
    pred, label, direc, xs2d, scomb = _p1a_call(
        hsumT, xsp.reshape(1, M), s0p[0].reshape(1, M), s0p[1].reshape(1, M),
        solp, wl, w2)
    preds = [pred[0, :N]]
    labels = [label[0, :N]]
    for t in range(1, 4):
        ppad = _p2(proj_matrix, direc[0, :N])
        qp = _sc_segsum()(ppad, src2, dst2, zvec)
        pred, label, direc, xs2d, scomb = _p1b_calls[t](
            hsumT, xs2d, scomb, qp[0].reshape(1, M), qp[1].reshape(1, M),
            ppad.reshape(1, M), solp, wl, w2)
        preds.append(pred[0, :N])
        labels.append(label[0, :N])
    return jnp.stack(preds, 1), jnp.stack(labels, 1)


# speedup vs baseline: 1.4419x; 1.0361x over previous
import functools

import jax
import jax.numpy as jnp
from jax import lax
from jax.experimental import pallas as pl
from jax.experimental.pallas import tpu as pltpu
from jax.experimental.pallas import tpu_sc as plsc

N = 10000
E = 320000
DF = 128
M = 10240
NCORE = 2
NSUB = 16
NT = NCORE * NSUB
EP = E // NT
CS = 80
NCS = EP // CS
ROWS_PER_TILE = M // NSUB


CA = 125
NCA = EP // CA


def _sc_row_aggregate_body(x_hbm, src_hbm, dst_hbm, zeros_hbm, out_hbm,
                           src_v, dst_v, rows_v, acc_sh, sem):
    c = lax.axis_index("c")
    s = lax.axis_index("s")
    w = c * NSUB + s

    @pl.when(s == 0)
    def _():
        pltpu.sync_copy(zeros_hbm, acc_sh)

    pltpu.sync_copy(src_hbm.at[w], src_v)
    pltpu.sync_copy(dst_hbm.at[w], dst_v)
    plsc.subcore_barrier()

    def body(j, carry):
        pltpu.async_copy(x_hbm.at[src_v.at[j]], rows_v, sem).wait()
        pltpu.sync_copy(rows_v, acc_sh.at[dst_v.at[j]], add=True)
        return carry

    lax.fori_loop(0, NCA, body, 0, unroll=False)
    plsc.subcore_barrier()
    pltpu.sync_copy(acc_sh.at[pl.ds(s * ROWS_PER_TILE, ROWS_PER_TILE)],
                    out_hbm.at[c, pl.ds(s * ROWS_PER_TILE, ROWS_PER_TILE)])


@functools.lru_cache(maxsize=None)
def _sc_row_aggregate():
    mesh = plsc.VectorSubcoreMesh(core_axis_name="c", subcore_axis_name="s",
                                  num_cores=NCORE, num_subcores=NSUB)
    return pl.kernel(
        _sc_row_aggregate_body,
        out_type=jax.ShapeDtypeStruct((NCORE, M, DF), jnp.float32),
        mesh=mesh,
        scratch_types=[
            pltpu.VMEM((NCA, CA), jnp.int32),
            pltpu.VMEM((NCA, CA), jnp.int32),
            pltpu.VMEM((CA, DF), jnp.float32),
            pltpu.VMEM_SHARED((M, DF), jnp.float32),
            pltpu.SemaphoreType.DMA,
        ],
        compiler_params=pltpu.CompilerParams(needs_layout_passes=False),
    )


def _sc_segsum_body(xs_hbm, src_hbm, dst_hbm, zeros_hbm, out_hbm,
                    xs_v, src_v, dst_v, vals_v, acc_sh, sem):
    c = lax.axis_index("c")
    s = lax.axis_index("s")
    w = c * NSUB + s

    @pl.when(s == 0)
    def _():
        pltpu.sync_copy(zeros_hbm, acc_sh)

    pltpu.sync_copy(xs_hbm, xs_v)
    pltpu.sync_copy(src_hbm.at[w], src_v)
    pltpu.sync_copy(dst_hbm.at[w], dst_v)
    plsc.subcore_barrier()

    def gbody(j, carry):
        for k in range(CS // 16):
            idx = src_v[j, pl.ds(k * 16, 16)]
            vals_v[pl.ds(j * CS + k * 16, 16)] = plsc.load_gather(xs_v, [idx])
        return carry

    lax.fori_loop(0, NCS, gbody, 0, unroll=False)

    FIRE = 5

    def sbody(jo, carry):
        base = jo * FIRE
        cps = [
            pltpu.async_copy(vals_v.at[pl.ds((base + k) * CS, CS)],
                             acc_sh.at[dst_v.at[base + k]], sem, add=True)
            for k in range(FIRE)
        ]
        for cp in cps:
            cp.wait()
        return carry

    lax.fori_loop(0, NCS // FIRE, sbody, 0, unroll=False)
    plsc.subcore_barrier()
    pltpu.sync_copy(acc_sh.at[pl.ds(s * ROWS_PER_TILE, ROWS_PER_TILE)],
                    out_hbm.at[c, pl.ds(s * ROWS_PER_TILE, ROWS_PER_TILE)])


@functools.lru_cache(maxsize=None)
def _sc_segsum():
    mesh = plsc.VectorSubcoreMesh(core_axis_name="c", subcore_axis_name="s",
                                  num_cores=NCORE, num_subcores=NSUB)
    return pl.kernel(
        _sc_segsum_body,
        out_type=jax.ShapeDtypeStruct((NCORE, M), jnp.float32),
        mesh=mesh,
        scratch_types=[
            pltpu.VMEM((M,), jnp.float32),
            pltpu.VMEM((NCS, CS), jnp.int32),
            pltpu.VMEM((NCS, CS), jnp.int32),
            pltpu.VMEM((EP,), jnp.float32),
            pltpu.VMEM_SHARED((M,), jnp.float32),
            pltpu.SemaphoreType.DMA,
        ],
        compiler_params=pltpu.CompilerParams(needs_layout_passes=False),
    )


def _p0_body(w1p_ref, x_ref, a0_ref, a1_ref, o_ref):
    xsum = x_ref[...] + a0_ref[...] + a1_ref[...]
    o_ref[...] = lax.dot_general(
        w1p_ref[...], xsum, (((0,), (1,)), ((), ())),
        preferred_element_type=jnp.float32, precision=lax.Precision.HIGHEST)


NB0 = 2048
_p0 = pl.pallas_call(
    _p0_body,
    grid=(M // NB0,),
    in_specs=[
        pl.BlockSpec((DF, DF), lambda i: (0, 0)),
        pl.BlockSpec((NB0, DF), lambda i: (i, 0)),
        pl.BlockSpec((NB0, DF), lambda i: (i, 0)),
        pl.BlockSpec((NB0, DF), lambda i: (i, 0)),
    ],
    out_specs=pl.BlockSpec((DF, NB0), lambda i: (0, i)),
    out_shape=jax.ShapeDtypeStruct((DF, M), jnp.float32),
)


def _finish_step(tau, hsumT, xs, sarr, sol, wl, w2,
                 pred_ref, label_ref, dir_ref, xs_ref, s_ref):
    col = lax.broadcasted_iota(jnp.int32, (1, M), 1)
    u = xs + sarr
    zt = jnp.tanh(hsumT + wl * u)
    pred = jnp.sum(zt * w2, axis=0, keepdims=True)
    pred = jnp.where(col < N, pred, 0.0)
    t1 = jnp.sum(jnp.abs(pred))
    res = sol - xs
    t2 = jnp.sum(jnp.abs(res))
    pred_ref[...] = pred
    label_ref[...] = res / (t2 + 1e-8)
    dir_ref[...] = pred / (t1 + 1e-8) + (3.0 * tau) / (xs + tau)
    xs_ref[...] = xs
    s_ref[...] = sarr


def _p1a_body(tau, hsumT_ref, xs_ref, s0_ref, s1_ref, sol_ref, wl_ref,
              w2_ref, pred_o, label_o, dir_o, xs_o, s_o):
    _finish_step(tau, hsumT_ref[...], xs_ref[...], s0_ref[...] + s1_ref[...],
                 sol_ref[...], wl_ref[...], w2_ref[...],
                 pred_o, label_o, dir_o, xs_o, s_o)


def _p1b_body(tau, hsumT_ref, xsp_ref, sprev_ref, q0_ref, q1_ref, pp_ref,
              sol_ref, wl_ref, w2_ref, pred_o, label_o, dir_o, xs_o, s_o):
    col = lax.broadcasted_iota(jnp.int32, (1, M), 1)
    valid = col < N
    xsp = xsp_ref[...]
    pp = pp_ref[...]
    ratios = jnp.where(valid & (pp < 0.0),
                       xsp / jnp.maximum(-pp, 1e-12), jnp.inf)
    alpha = jnp.minimum(jnp.min(ratios), STEP_ALPHA_MAX) * 0.995
    xs = jnp.where(valid, xsp + alpha * pp, 0.0)
    sarr = sprev_ref[...] + alpha * (q0_ref[...] + q1_ref[...])
    _finish_step(tau, hsumT_ref[...], xs, sarr, sol_ref[...], wl_ref[...],
                 w2_ref[...], pred_o, label_o, dir_o, xs_o, s_o)


STEP_ALPHA_MAX = 5.0
_P1_OUT = [jax.ShapeDtypeStruct((1, M), jnp.float32)] * 5


def _make_p1a(tau):
    return pl.pallas_call(functools.partial(_p1a_body, tau),
                          out_shape=_P1_OUT)


def _make_p1b(tau):
    return pl.pallas_call(functools.partial(_p1b_body, tau),
                          out_shape=_P1_OUT)


_TAUS = []
_t = 0.01
for _ in range(4):
    _TAUS.append(_t)
    _t = max(_t * 0.5, 1e-5)
_p1a_call = _make_p1a(_TAUS[0])
_p1b_calls = [None] + [_make_p1b(t) for t in _TAUS[1:]]


def _p2_body(p_ref, d_ref, o_ref):
    o_ref[...] = lax.dot_general(
        p_ref[...], d_ref[...], (((1,), (0,)), ((), ())),
        preferred_element_type=jnp.float32, precision=lax.Precision.HIGHEST)


RB = 512
_p2 = pl.pallas_call(
    _p2_body,
    grid=(M // RB,),
    in_specs=[
        pl.BlockSpec((RB, N), lambda i: (i, 0)),
        pl.BlockSpec((N,), lambda i: (0,)),
    ],
    out_specs=pl.BlockSpec((RB,), lambda i: (i,)),
    out_shape=jax.ShapeDtypeStruct((M,), jnp.float32),
)


def kernel(x, x_start, x_solution, proj_matrix, W1, W2, edge_index, vals_batch):
    del vals_batch
    f32 = jnp.float32
    x = x.astype(f32)
    xp = jnp.pad(x, ((0, M - N), (0, 0)))
    xsp = jnp.pad(x_start.astype(f32), (0, M - N))
    solp = jnp.pad(x_solution.astype(f32), (0, M - N)).reshape(1, M)
    W1p = W1[:DF]
    wl = W1[DF].reshape(DF, 1)
    w2 = W2.reshape(DF, 1)
    src2 = edge_index[0].reshape(NT, NCS, CS)
    dst2 = edge_index[1].reshape(NT, NCS, CS)
    zrows = jnp.zeros((M, DF), f32)
    zvec = jnp.zeros((M,), f32)

    src2a = edge_index[0].reshape(NT, NCA, CA)
    dst2a = edge_index[1].reshape(NT, NCA, CA)
    ax = _sc_row_aggregate()(x, src2a, dst2a, zrows)
    hsumT = _p0(W1p, xp, ax[0], ax[1])

    s0p = _sc_segsum()(xsp, src2, dst2, zvec)
    pred, label, direc, xs2d, scomb = _p1a_call(
        hsumT, xsp.reshape(1, M), s0p[0].reshape(1, M), s0p[1].reshape(1, M),
        solp, wl, w2)
    preds = [pred[0, :N]]
    labels = [label[0, :N]]
    for t in range(1, 4):
        ppad = _p2(proj_matrix, direc[0, :N])
        qp = _sc_segsum()(ppad, src2, dst2, zvec)
        pred, label, direc, xs2d, scomb = _p1b_calls[t](
            hsumT, xs2d, scomb, qp[0].reshape(1, M), qp[1].reshape(1, M),
            ppad.reshape(1, M), solp, wl, w2)
        preds.append(pred[0, :N])
        labels.append(label[0, :N])
    return jnp.stack(preds, 1), jnp.stack(labels, 1)


# speedup vs baseline: 1.4500x; 1.0056x over previous
import functools

import jax
import jax.numpy as jnp
from jax import lax
from jax.experimental import pallas as pl
from jax.experimental.pallas import tpu as pltpu
from jax.experimental.pallas import tpu_sc as plsc

N = 10000
E = 320000
DF = 128
M = 10240
NCORE = 2
NSUB = 16
NT = NCORE * NSUB
EP = E // NT
CS = 80
NCS = EP // CS
ROWS_PER_TILE = M // NSUB


CA = 125
NCA = EP // CA


def _sc_row_aggregate_body(x_hbm, src_hbm, dst_hbm, zeros_hbm, out_hbm,
                           src_v, dst_v, rows_v, acc_sh, sem):
    c = lax.axis_index("c")
    s = lax.axis_index("s")
    w = c * NSUB + s

    @pl.when(s == 0)
    def _():
        pltpu.sync_copy(zeros_hbm, acc_sh)

    pltpu.sync_copy(src_hbm.at[w], src_v)
    pltpu.sync_copy(dst_hbm.at[w], dst_v)
    plsc.subcore_barrier()

    def body(j, carry):
        pltpu.async_copy(x_hbm.at[src_v.at[j]], rows_v, sem).wait()
        pltpu.sync_copy(rows_v, acc_sh.at[dst_v.at[j]], add=True)
        return carry

    lax.fori_loop(0, NCA, body, 0, unroll=False)
    plsc.subcore_barrier()
    pltpu.sync_copy(acc_sh.at[pl.ds(s * ROWS_PER_TILE, ROWS_PER_TILE)],
                    out_hbm.at[c, pl.ds(s * ROWS_PER_TILE, ROWS_PER_TILE)])


@functools.lru_cache(maxsize=None)
def _sc_row_aggregate():
    mesh = plsc.VectorSubcoreMesh(core_axis_name="c", subcore_axis_name="s",
                                  num_cores=NCORE, num_subcores=NSUB)
    return pl.kernel(
        _sc_row_aggregate_body,
        out_type=jax.ShapeDtypeStruct((NCORE, M, DF), jnp.float32),
        mesh=mesh,
        scratch_types=[
            pltpu.VMEM((NCA, CA), jnp.int32),
            pltpu.VMEM((NCA, CA), jnp.int32),
            pltpu.VMEM((CA, DF), jnp.float32),
            pltpu.VMEM_SHARED((M, DF), jnp.float32),
            pltpu.SemaphoreType.DMA,
        ],
        compiler_params=pltpu.CompilerParams(needs_layout_passes=False),
    )


def _sc_segsum_body(xs_hbm, src_hbm, dst_hbm, zeros_hbm, out_hbm,
                    xs_v, src_v, dst_v, vals_v, acc_sh, sem):
    c = lax.axis_index("c")
    s = lax.axis_index("s")
    w = c * NSUB + s

    @pl.when(s == 0)
    def _():
        pltpu.sync_copy(zeros_hbm, acc_sh)

    pltpu.sync_copy(xs_hbm, xs_v)
    pltpu.sync_copy(src_hbm.at[w], src_v)
    pltpu.sync_copy(dst_hbm.at[w], dst_v)
    plsc.subcore_barrier()

    def gbody(j, carry):
        for k in range(CS // 16):
            idx = src_v[j, pl.ds(k * 16, 16)]
            vals_v[pl.ds(j * CS + k * 16, 16)] = plsc.load_gather(xs_v, [idx])
        return carry

    lax.fori_loop(0, NCS, gbody, 0, unroll=False)

    FIRE = 25

    def sbody(jo, carry):
        base = jo * FIRE
        cps = [
            pltpu.async_copy(vals_v.at[pl.ds((base + k) * CS, CS)],
                             acc_sh.at[dst_v.at[base + k]], sem, add=True)
            for k in range(FIRE)
        ]
        for cp in cps:
            cp.wait()
        return carry

    lax.fori_loop(0, NCS // FIRE, sbody, 0, unroll=False)
    plsc.subcore_barrier()
    pltpu.sync_copy(acc_sh.at[pl.ds(s * ROWS_PER_TILE, ROWS_PER_TILE)],
                    out_hbm.at[c, pl.ds(s * ROWS_PER_TILE, ROWS_PER_TILE)])


@functools.lru_cache(maxsize=None)
def _sc_segsum():
    mesh = plsc.VectorSubcoreMesh(core_axis_name="c", subcore_axis_name="s",
                                  num_cores=NCORE, num_subcores=NSUB)
    return pl.kernel(
        _sc_segsum_body,
        out_type=jax.ShapeDtypeStruct((NCORE, M), jnp.float32),
        mesh=mesh,
        scratch_types=[
            pltpu.VMEM((M,), jnp.float32),
            pltpu.VMEM((NCS, CS), jnp.int32),
            pltpu.VMEM((NCS, CS), jnp.int32),
            pltpu.VMEM((EP,), jnp.float32),
            pltpu.VMEM_SHARED((M,), jnp.float32),
            pltpu.SemaphoreType.DMA,
        ],
        compiler_params=pltpu.CompilerParams(needs_layout_passes=False),
    )


def _p0_body(w1p_ref, x_ref, a0_ref, a1_ref, o_ref):
    xsum = x_ref[...] + a0_ref[...] + a1_ref[...]
    o_ref[...] = lax.dot_general(
        w1p_ref[...], xsum, (((0,), (1,)), ((), ())),
        preferred_element_type=jnp.float32, precision=lax.Precision.HIGHEST)


NB0 = 2048
_p0 = pl.pallas_call(
    _p0_body,
    grid=(M // NB0,),
    in_specs=[
        pl.BlockSpec((DF, DF), lambda i: (0, 0)),
        pl.BlockSpec((NB0, DF), lambda i: (i, 0)),
        pl.BlockSpec((NB0, DF), lambda i: (i, 0)),
        pl.BlockSpec((NB0, DF), lambda i: (i, 0)),
    ],
    out_specs=pl.BlockSpec((DF, NB0), lambda i: (0, i)),
    out_shape=jax.ShapeDtypeStruct((DF, M), jnp.float32),
)


def _finish_step(tau, hsumT, xs, sarr, sol, wl, w2,
                 pred_ref, label_ref, dir_ref, xs_ref, s_ref):
    col = lax.broadcasted_iota(jnp.int32, (1, M), 1)
    u = xs + sarr
    zt = jnp.tanh(hsumT + wl * u)
    pred = jnp.sum(zt * w2, axis=0, keepdims=True)
    pred = jnp.where(col < N, pred, 0.0)
    t1 = jnp.sum(jnp.abs(pred))
    res = sol - xs
    t2 = jnp.sum(jnp.abs(res))
    pred_ref[...] = pred
    label_ref[...] = res / (t2 + 1e-8)
    dir_ref[...] = pred / (t1 + 1e-8) + (3.0 * tau) / (xs + tau)
    xs_ref[...] = xs
    s_ref[...] = sarr


def _p1a_body(tau, hsumT_ref, xs_ref, s0_ref, s1_ref, sol_ref, wl_ref,
              w2_ref, pred_o, label_o, dir_o, xs_o, s_o):
    _finish_step(tau, hsumT_ref[...], xs_ref[...], s0_ref[...] + s1_ref[...],
                 sol_ref[...], wl_ref[...], w2_ref[...],
                 pred_o, label_o, dir_o, xs_o, s_o)


def _p1b_body(tau, hsumT_ref, xsp_ref, sprev_ref, q0_ref, q1_ref, pp_ref,
              sol_ref, wl_ref, w2_ref, pred_o, label_o, dir_o, xs_o, s_o):
    col = lax.broadcasted_iota(jnp.int32, (1, M), 1)
    valid = col < N
    xsp = xsp_ref[...]
    pp = pp_ref[...]
    ratios = jnp.where(valid & (pp < 0.0),
                       xsp / jnp.maximum(-pp, 1e-12), jnp.inf)
    alpha = jnp.minimum(jnp.min(ratios), STEP_ALPHA_MAX) * 0.995
    xs = jnp.where(valid, xsp + alpha * pp, 0.0)
    sarr = sprev_ref[...] + alpha * (q0_ref[...] + q1_ref[...])
    _finish_step(tau, hsumT_ref[...], xs, sarr, sol_ref[...], wl_ref[...],
                 w2_ref[...], pred_o, label_o, dir_o, xs_o, s_o)


STEP_ALPHA_MAX = 5.0
_P1_OUT = [jax.ShapeDtypeStruct((1, M), jnp.float32)] * 5


def _make_p1a(tau):
    return pl.pallas_call(functools.partial(_p1a_body, tau),
                          out_shape=_P1_OUT)


def _make_p1b(tau):
    return pl.pallas_call(functools.partial(_p1b_body, tau),
                          out_shape=_P1_OUT)


_TAUS = []
_t = 0.01
for _ in range(4):
    _TAUS.append(_t)
    _t = max(_t * 0.5, 1e-5)
_p1a_call = _make_p1a(_TAUS[0])
_p1b_calls = [None] + [_make_p1b(t) for t in _TAUS[1:]]


def _p2_body(p_ref, d_ref, o_ref):
    o_ref[...] = lax.dot_general(
        p_ref[...], d_ref[...], (((1,), (0,)), ((), ())),
        preferred_element_type=jnp.float32, precision=lax.Precision.HIGHEST)


RB = 512
_p2 = pl.pallas_call(
    _p2_body,
    grid=(M // RB,),
    in_specs=[
        pl.BlockSpec((RB, N), lambda i: (i, 0)),
        pl.BlockSpec((N,), lambda i: (0,)),
    ],
    out_specs=pl.BlockSpec((RB,), lambda i: (i,)),
    out_shape=jax.ShapeDtypeStruct((M,), jnp.float32),
)


def kernel(x, x_start, x_solution, proj_matrix, W1, W2, edge_index, vals_batch):
    del vals_batch
    f32 = jnp.float32
    x = x.astype(f32)
    xp = jnp.pad(x, ((0, M - N), (0, 0)))
    xsp = jnp.pad(x_start.astype(f32), (0, M - N))
    solp = jnp.pad(x_solution.astype(f32), (0, M - N)).reshape(1, M)
    W1p = W1[:DF]
    wl = W1[DF].reshape(DF, 1)
    w2 = W2.reshape(DF, 1)
    src2 = edge_index[0].reshape(NT, NCS, CS)
    dst2 = edge_index[1].reshape(NT, NCS, CS)
    zrows = jnp.zeros((M, DF), f32)
    zvec = jnp.zeros((M,), f32)

    src2a = edge_index[0].reshape(NT, NCA, CA)
    dst2a = edge_index[1].reshape(NT, NCA, CA)
    ax = _sc_row_aggregate()(x, src2a, dst2a, zrows)
    hsumT = _p0(W1p, xp, ax[0], ax[1])

    s0p = _sc_segsum()(xsp, src2, dst2, zvec)
    pred, label, direc, xs2d, scomb = _p1a_call(
        hsumT, xsp.reshape(1, M), s0p[0].reshape(1, M), s0p[1].reshape(1, M),
        solp, wl, w2)
    preds = [pred[0, :N]]
    labels = [label[0, :N]]
    for t in range(1, 4):
        ppad = _p2(proj_matrix, direc[0, :N])
        qp = _sc_segsum()(ppad, src2, dst2, zvec)
        pred, label, direc, xs2d, scomb = _p1b_calls[t](
            hsumT, xs2d, scomb, qp[0].reshape(1, M), qp[1].reshape(1, M),
            ppad.reshape(1, M), solp, wl, w2)
        preds.append(pred[0, :N])
        labels.append(label[0, :N])
    return jnp.stack(preds, 1), jnp.stack(labels, 1)
